# Initial kernel scaffold; baseline (speedup 1.0000x reference)
import jax
import jax.numpy as jnp
from jax.experimental import pallas as pl


def kernel(feat_embs, position_ids, segment_ids, pos_table, seg_table, ln_gamma, ln_beta):
    raise NotImplementedError("write your pallas kernel here")


# speedup vs baseline: 1.5661x; 1.5661x over previous
import functools

import jax
import jax.numpy as jnp
from jax import lax
from jax.experimental import pallas as pl
from jax.experimental.pallas import tpu as pltpu
from jax.experimental.pallas import tpu_sc as plsc

_LN_EPS = 1e-12


def _sc_gather(table, idx):
    V, D = table.shape
    N = idx.shape[0]
    info = plsc.get_sparse_core_info()
    NC, NS = info.num_cores, info.num_subcores
    NW = NC * NS
    assert N % NW == 0
    b_per_w = N // NW
    CH = 128 if b_per_w % 128 == 0 else b_per_w
    n_ch = b_per_w // CH
    mesh = plsc.VectorSubcoreMesh(core_axis_name="c", subcore_axis_name="s")

    @functools.partial(
        pl.kernel,
        mesh=mesh,
        out_type=jax.ShapeDtypeStruct((N, D), jnp.float32),
        scratch_types=[
            pltpu.VMEM((b_per_w,), jnp.int32),
            pltpu.VMEM((CH, D), jnp.float32),
            pltpu.SemaphoreType.DMA,
        ],
    )
    def k(table_hbm, idx_hbm, out_hbm, idx_v, rows_v, sem):
        wid = lax.axis_index("s") * NC + lax.axis_index("c")
        base = wid * b_per_w
        pltpu.sync_copy(idx_hbm.at[pl.ds(base, b_per_w)], idx_v)
        for j in range(n_ch):
            pltpu.async_copy(table_hbm.at[idx_v.at[pl.ds(j * CH, CH)]],
                             rows_v, sem).wait()
            pltpu.sync_copy(rows_v, out_hbm.at[pl.ds(base + j * CH, CH)])

    return k(table, idx)


def _tc_fused_ln(feat2, pos_rows, sidf, seg_table, gamma2, beta2):
    N, D = feat2.shape
    BT = 256

    def body(f_ref, p_ref, sid_ref, seg_ref, g_ref, b_ref, o_ref):
        x = f_ref[...] + p_ref[...]
        seg0 = seg_ref[0:1, :]
        dseg = seg_ref[1:2, :] - seg0
        x = x + seg0 + sid_ref[...] * dseg
        mean = jnp.mean(x, axis=1, keepdims=True)
        xc = x - mean
        var = jnp.mean(xc * xc, axis=1, keepdims=True)
        rstd = lax.rsqrt(var + _LN_EPS)
        o_ref[...] = xc * rstd * g_ref[...] + b_ref[...]

    return pl.pallas_call(
        body,
        grid=(N // BT,),
        in_specs=[
            pl.BlockSpec((BT, D), lambda i: (i, 0)),
            pl.BlockSpec((BT, D), lambda i: (i, 0)),
            pl.BlockSpec((BT, 1), lambda i: (i, 0)),
            pl.BlockSpec((2, D), lambda i: (0, 0)),
            pl.BlockSpec((1, D), lambda i: (0, 0)),
            pl.BlockSpec((1, D), lambda i: (0, 0)),
        ],
        out_specs=pl.BlockSpec((BT, D), lambda i: (i, 0)),
        out_shape=jax.ShapeDtypeStruct((N, D), jnp.float32),
    )(feat2, pos_rows, sidf, seg_table, gamma2, beta2)


def kernel(feat_embs, position_ids, segment_ids, pos_table, seg_table,
           ln_gamma, ln_beta):
    B, S, D = feat_embs.shape
    N = B * S
    feat2 = feat_embs.reshape(N, D)
    pos = position_ids.reshape(N).astype(jnp.int32)
    sidf = segment_ids.reshape(N, 1).astype(jnp.float32)
    pos_rows = _sc_gather(pos_table.astype(jnp.float32), pos)
    out2 = _tc_fused_ln(feat2, pos_rows, sidf,
                        seg_table.astype(jnp.float32),
                        ln_gamma.reshape(1, D), ln_beta.reshape(1, D))
    return out2.reshape(B, S, D)


# speedup vs baseline: 1.8242x; 1.1649x over previous
import functools

import jax
import jax.numpy as jnp
from jax import lax
from jax.experimental import pallas as pl
from jax.experimental.pallas import tpu as pltpu
from jax.experimental.pallas import tpu_sc as plsc

_LN_EPS = 1e-12


def _sc_gather(table, idx):
    V, D = table.shape
    N = idx.shape[0]
    info = plsc.get_sparse_core_info()
    NC, NS = info.num_cores, info.num_subcores
    NW = NC * NS
    assert N % NW == 0
    b_per_w = N // NW
    CH = 128 if b_per_w % 128 == 0 else b_per_w
    n_ch = b_per_w // CH
    mesh = plsc.VectorSubcoreMesh(core_axis_name="c", subcore_axis_name="s")

    @functools.partial(
        pl.kernel,
        mesh=mesh,
        out_type=jax.ShapeDtypeStruct((N, D), jnp.float32),
        scratch_types=[
            pltpu.VMEM((b_per_w,), jnp.int32),
            pltpu.VMEM((CH, D), jnp.float32),
            pltpu.SemaphoreType.DMA,
        ],
    )
    def k(table_hbm, idx_hbm, out_hbm, idx_v, rows_v, sem):
        wid = lax.axis_index("s") * NC + lax.axis_index("c")
        base = wid * b_per_w
        pltpu.sync_copy(idx_hbm.at[pl.ds(base, b_per_w)], idx_v)
        for j in range(n_ch):
            pltpu.async_copy(table_hbm.at[idx_v.at[pl.ds(j * CH, CH)]],
                             rows_v, sem).wait()
            pltpu.sync_copy(rows_v, out_hbm.at[pl.ds(base + j * CH, CH)])

    return k(table, idx)


def _tc_fused_ln(feat2, pos_rows, sidf, seg_table, gamma2, beta2):
    N, D = feat2.shape
    BT = 512

    def body(f_ref, p_ref, sid_ref, seg_ref, g_ref, b_ref, o_ref):
        x = f_ref[...] + p_ref[...]
        seg0 = seg_ref[0:1, :]
        dseg = seg_ref[1:2, :] - seg0
        x = x + seg0 + sid_ref[...] * dseg
        mean = jnp.mean(x, axis=1, keepdims=True)
        xc = x - mean
        var = jnp.mean(xc * xc, axis=1, keepdims=True)
        rstd = lax.rsqrt(var + _LN_EPS)
        o_ref[...] = xc * rstd * g_ref[...] + b_ref[...]

    return pl.pallas_call(
        body,
        grid=(N // BT,),
        in_specs=[
            pl.BlockSpec((BT, D), lambda i: (i, 0)),
            pl.BlockSpec((BT, D), lambda i: (i, 0)),
            pl.BlockSpec((BT, 1), lambda i: (i, 0)),
            pl.BlockSpec((2, D), lambda i: (0, 0)),
            pl.BlockSpec((1, D), lambda i: (0, 0)),
            pl.BlockSpec((1, D), lambda i: (0, 0)),
        ],
        out_specs=pl.BlockSpec((BT, D), lambda i: (i, 0)),
        out_shape=jax.ShapeDtypeStruct((N, D), jnp.float32),
        compiler_params=pltpu.CompilerParams(
            dimension_semantics=("arbitrary",)),
    )(feat2, pos_rows, sidf, seg_table, gamma2, beta2)


def kernel(feat_embs, position_ids, segment_ids, pos_table, seg_table,
           ln_gamma, ln_beta):
    B, S, D = feat_embs.shape
    N = B * S
    feat2 = feat_embs.reshape(N, D)
    pos = position_ids.reshape(N).astype(jnp.int32)
    sidf = segment_ids.reshape(N, 1).astype(jnp.float32)
    pos_rows = _sc_gather(pos_table.astype(jnp.float32), pos)
    out2 = _tc_fused_ln(feat2, pos_rows, sidf,
                        seg_table.astype(jnp.float32),
                        ln_gamma.reshape(1, D), ln_beta.reshape(1, D))
    return out2.reshape(B, S, D)


# speedup vs baseline: 1.8956x; 1.0391x over previous
import functools

import jax
import jax.numpy as jnp
from jax import lax
from jax.experimental import pallas as pl
from jax.experimental.pallas import tpu as pltpu
from jax.experimental.pallas import tpu_sc as plsc

_LN_EPS = 1e-12


def _sc_gather(table, idx):
    V, D = table.shape
    N = idx.shape[0]
    info = plsc.get_sparse_core_info()
    NC, NS = info.num_cores, info.num_subcores
    NW = NC * NS
    assert N % NW == 0
    b_per_w = N // NW
    CH = 128 if b_per_w % 128 == 0 else b_per_w
    n_ch = b_per_w // CH
    mesh = plsc.VectorSubcoreMesh(core_axis_name="c", subcore_axis_name="s")

    @functools.partial(
        pl.kernel,
        mesh=mesh,
        out_type=jax.ShapeDtypeStruct((N, D), jnp.float32),
        scratch_types=[
            pltpu.VMEM((b_per_w,), jnp.int32),
            pltpu.VMEM((CH, D), jnp.float32),
            pltpu.SemaphoreType.DMA,
        ],
    )
    def k(table_hbm, idx_hbm, out_hbm, idx_v, rows_v, sem):
        wid = lax.axis_index("s") * NC + lax.axis_index("c")
        base = wid * b_per_w
        pltpu.sync_copy(idx_hbm.at[pl.ds(base, b_per_w)], idx_v)
        for j in range(n_ch):
            pltpu.async_copy(table_hbm.at[idx_v.at[pl.ds(j * CH, CH)]],
                             rows_v, sem).wait()
            pltpu.sync_copy(rows_v, out_hbm.at[pl.ds(base + j * CH, CH)])

    return k(table, idx)


def _tc_fused_ln(feat2, pos_rows, sidf, seg_table, gamma2, beta2):
    N, D = feat2.shape
    BT = 1024

    def body(f_ref, p_ref, sid_ref, seg_ref, g_ref, b_ref, o_ref):
        x = f_ref[...] + p_ref[...]
        seg0 = seg_ref[0:1, :]
        dseg = seg_ref[1:2, :] - seg0
        x = x + seg0 + sid_ref[...] * dseg
        mean = jnp.mean(x, axis=1, keepdims=True)
        xc = x - mean
        var = jnp.mean(xc * xc, axis=1, keepdims=True)
        rstd = lax.rsqrt(var + _LN_EPS)
        o_ref[...] = xc * rstd * g_ref[...] + b_ref[...]

    return pl.pallas_call(
        body,
        grid=(N // BT,),
        in_specs=[
            pl.BlockSpec((BT, D), lambda i: (i, 0)),
            pl.BlockSpec((BT, D), lambda i: (i, 0)),
            pl.BlockSpec((BT, 1), lambda i: (i, 0)),
            pl.BlockSpec((2, D), lambda i: (0, 0)),
            pl.BlockSpec((1, D), lambda i: (0, 0)),
            pl.BlockSpec((1, D), lambda i: (0, 0)),
        ],
        out_specs=pl.BlockSpec((BT, D), lambda i: (i, 0)),
        out_shape=jax.ShapeDtypeStruct((N, D), jnp.float32),
        compiler_params=pltpu.CompilerParams(
            dimension_semantics=("arbitrary",)),
    )(feat2, pos_rows, sidf, seg_table, gamma2, beta2)


def kernel(feat_embs, position_ids, segment_ids, pos_table, seg_table,
           ln_gamma, ln_beta):
    B, S, D = feat_embs.shape
    N = B * S
    feat2 = feat_embs.reshape(N, D)
    pos = position_ids.reshape(N).astype(jnp.int32)
    sidf = segment_ids.reshape(N, 1).astype(jnp.float32)
    pos_rows = _sc_gather(pos_table.astype(jnp.float32), pos)
    out2 = _tc_fused_ln(feat2, pos_rows, sidf,
                        seg_table.astype(jnp.float32),
                        ln_gamma.reshape(1, D), ln_beta.reshape(1, D))
    return out2.reshape(B, S, D)
